# speedup vs baseline: 3.2977x; 3.2977x over previous
import functools

import jax
import jax.numpy as jnp
from jax import lax
from jax.experimental import pallas as pl
from jax.experimental.pallas import tpu as pltpu
from jax.experimental.pallas import tpu_sc as plsc

D = 128
B = 4096 * 50
NC, NS = 2, 16
NW = NC * NS
PER_W = B // NW
CH = 128
NCHUNK = PER_W // CH
NBUF = 5
NG = NCHUNK // NBUF

_mesh = plsc.VectorSubcoreMesh(core_axis_name="c", subcore_axis_name="s")


@functools.partial(
    pl.kernel,
    mesh=_mesh,
    out_type=jax.ShapeDtypeStruct((B, D), jnp.float32),
    scratch_types=[
        pltpu.VMEM((NCHUNK, CH), jnp.int32),
        pltpu.VMEM((NBUF, CH, D), jnp.float32),
        pltpu.SemaphoreType.DMA((NBUF,)),
        pltpu.SemaphoreType.DMA((NBUF,)),
    ],
)
def _embed_gather(table_hbm, idx_hbm, out_hbm, idx_v, rows_v, gsem, osem):
    wid = lax.axis_index("s") * NC + lax.axis_index("c")
    base = wid * PER_W
    pltpu.sync_copy(idx_hbm.at[wid], idx_v)

    def _gather(b, c):
        pltpu.async_copy(table_hbm.at[idx_v.at[c]], rows_v.at[b], gsem.at[b])

    def _gather_wait(b):
        pltpu.make_async_copy(
            table_hbm.at[idx_v.at[0]], rows_v.at[b], gsem.at[b]).wait()

    def _write(b, c):
        pltpu.async_copy(
            rows_v.at[b], out_hbm.at[pl.ds(base + c * CH, CH)], osem.at[b])

    def _write_wait(b):
        pltpu.make_async_copy(
            rows_v.at[b], out_hbm.at[pl.ds(base, CH)], osem.at[b]).wait()

    for b in range(NBUF):
        _gather(b, b)

    def _group(g, carry):
        for b in range(NBUF):
            _gather_wait(b)
            _write(b, g * NBUF + b)
        for b in range(NBUF):
            _write_wait(b)
            _gather(b, (g + 1) * NBUF + b)
        return carry

    lax.fori_loop(0, NG - 1, _group, 0)

    for b in range(NBUF):
        _gather_wait(b)
        _write(b, (NG - 1) * NBUF + b)
    for b in range(NBUF):
        _write_wait(b)


def kernel(token_ids, weight):
    idx = token_ids.reshape(NW, NCHUNK, CH).astype(jnp.int32)
    out = _embed_gather(weight, idx)
    return out.reshape(token_ids.shape[0], token_ids.shape[1], D)


# speedup vs baseline: 5.1152x; 1.5511x over previous
import functools

import jax
import jax.numpy as jnp
from jax import lax
from jax.experimental import pallas as pl
from jax.experimental.pallas import tpu as pltpu
from jax.experimental.pallas import tpu_sc as plsc

D = 128
SEQ = 50
SEQP = 56
NBATCH = 4096
NC, NS = 2, 16
NW = NC * NS
NCHUNK = NBATCH // NW
NBUF = 4
NG = NCHUNK // NBUF

_mesh = plsc.VectorSubcoreMesh(core_axis_name="c", subcore_axis_name="s")


@functools.partial(
    pl.kernel,
    mesh=_mesh,
    out_type=jax.ShapeDtypeStruct((NBATCH, SEQ, D), jnp.float32),
    scratch_types=[
        pltpu.VMEM((NCHUNK, SEQP), jnp.int32),
        pltpu.VMEM((NBUF, SEQP, D), jnp.float32),
        pltpu.SemaphoreType.DMA((NBUF,)),
        pltpu.SemaphoreType.DMA((NBUF,)),
    ],
)
def _embed_gather(table_hbm, idx_hbm, out_hbm, idx_v, rows_v, gsem, osem):
    wid = lax.axis_index("s") * NC + lax.axis_index("c")
    base = wid * NCHUNK
    pltpu.sync_copy(idx_hbm.at[wid], idx_v)

    def _gather(b, c):
        pltpu.async_copy(table_hbm.at[idx_v.at[c]], rows_v.at[b], gsem.at[b])

    def _gather_wait(b):
        pltpu.make_async_copy(
            table_hbm.at[idx_v.at[0]], rows_v.at[b], gsem.at[b]).wait()

    def _write(b, c):
        pltpu.async_copy(
            rows_v.at[b, pl.ds(0, SEQ)], out_hbm.at[base + c], osem.at[b])

    def _write_wait(b):
        pltpu.make_async_copy(
            rows_v.at[b, pl.ds(0, SEQ)], out_hbm.at[base], osem.at[b]).wait()

    for b in range(NBUF):
        _gather(b, b)

    def _group(g, carry):
        for b in range(NBUF):
            _gather_wait(b)
            _write(b, g * NBUF + b)
        for b in range(NBUF):
            _write_wait(b)
            _gather(b, (g + 1) * NBUF + b)
        return carry

    lax.fori_loop(0, NG - 1, _group, 0)

    for b in range(NBUF):
        _gather_wait(b)
        _write(b, (NG - 1) * NBUF + b)
    for b in range(NBUF):
        _write_wait(b)


def kernel(token_ids, weight):
    idx = token_ids.reshape(NW, NCHUNK, SEQ).astype(jnp.int32)
    idx = jnp.pad(idx, ((0, 0), (0, 0), (0, SEQP - SEQ)), mode="edge")
    return _embed_gather(weight, idx)


# speedup vs baseline: 5.7565x; 1.1254x over previous
import functools

import jax
import jax.numpy as jnp
from jax import lax
from jax.experimental import pallas as pl
from jax.experimental.pallas import tpu as pltpu
from jax.experimental.pallas import tpu_sc as plsc

D = 128
SEQ = 50
W2 = 2 * SEQ
WP = 104
NBATCH = 4096
NC, NS = 2, 16
NW = NC * NS
NCHUNK = NBATCH // (2 * NW)
NBUF = 4
NG = NCHUNK // NBUF

_mesh = plsc.VectorSubcoreMesh(core_axis_name="c", subcore_axis_name="s")


@functools.partial(
    pl.kernel,
    mesh=_mesh,
    out_type=jax.ShapeDtypeStruct((NBATCH, SEQ, D), jnp.float32),
    scratch_types=[
        pltpu.VMEM((NCHUNK, WP), jnp.int32),
        pltpu.VMEM((NBUF, WP, D), jnp.float32),
        pltpu.SemaphoreType.DMA((NBUF,)),
        pltpu.SemaphoreType.DMA((NBUF,)),
    ],
)
def _embed_gather(table_hbm, idx_hbm, out_hbm, idx_v, rows_v, gsem, osem):
    wid = lax.axis_index("s") * NC + lax.axis_index("c")
    base = wid * 2 * NCHUNK
    pltpu.sync_copy(idx_hbm.at[wid], idx_v)

    def _gather(b, c):
        pltpu.async_copy(table_hbm.at[idx_v.at[c]], rows_v.at[b], gsem.at[b])

    def _gather_wait(b):
        pltpu.make_async_copy(
            table_hbm.at[idx_v.at[0]], rows_v.at[b], gsem.at[b]).wait()

    def _write(b, c):
        pltpu.async_copy(
            rows_v.at[b, pl.ds(0, SEQ)], out_hbm.at[base + 2 * c], osem.at[b])
        pltpu.async_copy(
            rows_v.at[b, pl.ds(SEQ, SEQ)], out_hbm.at[base + 2 * c + 1],
            osem.at[b])

    def _write_wait(b):
        pltpu.make_async_copy(
            rows_v.at[b, pl.ds(0, W2)], out_hbm.at[pl.ds(base, 2)],
            osem.at[b]).wait()

    for b in range(NBUF):
        _gather(b, b)

    def _group(g, carry):
        for b in range(NBUF):
            _gather_wait(b)
            _write(b, g * NBUF + b)
        for b in range(NBUF):
            _write_wait(b)
            _gather(b, (g + 1) * NBUF + b)
        return carry

    lax.fori_loop(0, NG - 1, _group, 0)

    for b in range(NBUF):
        _gather_wait(b)
        _write(b, (NG - 1) * NBUF + b)
    for b in range(NBUF):
        _write_wait(b)


def kernel(token_ids, weight):
    idx = token_ids.reshape(NW, NCHUNK, W2).astype(jnp.int32)
    idx = jnp.pad(idx, ((0, 0), (0, 0), (0, WP - W2)), mode="edge")
    return _embed_gather(weight, idx)
